# Initial kernel scaffold; baseline (speedup 1.0000x reference)
import functools

import jax
import jax.numpy as jnp
from jax import lax
from jax.experimental import pallas as pl
from jax.experimental.pallas import tpu as pltpu
from jax.experimental.pallas import tpu_sc as plsc

N = 10000
E = 320000
G = 64
D = 128

NPAD = 10240
WE = 125
NWIN = E // WE
RPT = NWIN // 32
RB = 1000


def _vsc_mesh():
    return plsc.VectorSubcoreMesh(core_axis_name="c", subcore_axis_name="s")


def _sc_degree(col2d):
    @functools.partial(
        pl.kernel,
        out_type=jax.ShapeDtypeStruct((2, NPAD, 16), jnp.float32),
        mesh=_vsc_mesh(),
        scratch_types=[
            pltpu.VMEM((RPT, WE), jnp.int32),
            pltpu.VMEM((WE, 16), jnp.float32),
            pltpu.VMEM((64, 16), jnp.float32),
            pltpu.VMEM_SHARED((NPAD, 16), jnp.float32),
        ],
    )
    def k(col_hbm, out_hbm, idx_v, ones_v, zbuf, accum):
        c = lax.axis_index("c")
        s = lax.axis_index("s")
        wid = s * 2 + c

        @pl.loop(0, WE)
        def _(i):
            ones_v[i, :] = jnp.full((16,), 1.0, jnp.float32)

        @pl.loop(0, 64)
        def _(i):
            zbuf[i, :] = jnp.zeros((16,), jnp.float32)

        @pl.loop(0, 10)
        def _(i):
            pltpu.sync_copy(zbuf, accum.at[pl.ds(s * 640 + i * 64, 64)])

        plsc.subcore_barrier()
        pltpu.sync_copy(col_hbm.at[pl.ds(wid * RPT, RPT)], idx_v)

        @pl.loop(0, RPT)
        def _(j):
            pltpu.sync_copy(ones_v, accum.at[idx_v.at[j]], add=True)

        plsc.subcore_barrier()
        pltpu.sync_copy(accum.at[pl.ds(s * 640, 640)],
                        out_hbm.at[c, pl.ds(s * 640, 640)])

    return k(col2d)


def _sc_scatter(y, row2d, col2d):
    @functools.partial(
        pl.kernel,
        out_type=jax.ShapeDtypeStruct((2, NPAD, D), jnp.float32),
        mesh=_vsc_mesh(),
        scratch_types=[
            pltpu.VMEM((RPT, WE), jnp.int32),
            pltpu.VMEM((RPT, WE), jnp.int32),
            pltpu.VMEM((WE, D), jnp.float32),
            pltpu.VMEM((64, D), jnp.float32),
            pltpu.VMEM_SHARED((NPAD, D), jnp.float32),
        ],
    )
    def k(y_hbm, row_hbm, col_hbm, out_hbm, row_v, col_v, gbuf, zbuf, accum):
        c = lax.axis_index("c")
        s = lax.axis_index("s")
        wid = s * 2 + c

        @pl.loop(0, 64)
        def _(i):
            @pl.loop(0, D // 16)
            def _(j):
                zbuf[i, pl.ds(j * 16, 16)] = jnp.zeros((16,), jnp.float32)

        @pl.loop(0, 10)
        def _(i):
            pltpu.sync_copy(zbuf, accum.at[pl.ds(s * 640 + i * 64, 64)])

        plsc.subcore_barrier()
        pltpu.sync_copy(row_hbm.at[pl.ds(wid * RPT, RPT)], row_v)
        pltpu.sync_copy(col_hbm.at[pl.ds(wid * RPT, RPT)], col_v)

        @pl.loop(0, RPT)
        def _(j):
            pltpu.sync_copy(y_hbm.at[row_v.at[j]], gbuf)
            pltpu.sync_copy(gbuf, accum.at[col_v.at[j]], add=True)

        plsc.subcore_barrier()
        pltpu.sync_copy(accum.at[pl.ds(s * 640, 640)],
                        out_hbm.at[c, pl.ds(s * 640, 640)])

    return k(y, row2d, col2d)


def _dis_kernel(d0_ref, d1_ref, o_ref):
    o_ref[...] = lax.rsqrt(1.0 + d0_ref[...] + d1_ref[...])


def _dis(d0, d1):
    return pl.pallas_call(
        _dis_kernel,
        grid=(N // RB,),
        in_specs=[pl.BlockSpec((RB, 1), lambda i: (i, 0)),
                  pl.BlockSpec((RB, 1), lambda i: (i, 0))],
        out_specs=pl.BlockSpec((RB, 1), lambda i: (i, 0)),
        out_shape=jax.ShapeDtypeStruct((N, 1), jnp.float32),
    )(d0, d1)


def _mm_scale_kernel(x_ref, w_ref, dis_ref, o_ref):
    h = jnp.dot(x_ref[...], w_ref[...], preferred_element_type=jnp.float32)
    o_ref[...] = h * dis_ref[...]


def _mm_scale(x, W, dis):
    return pl.pallas_call(
        _mm_scale_kernel,
        grid=(N // RB,),
        in_specs=[pl.BlockSpec((RB, D), lambda i: (i, 0)),
                  pl.BlockSpec((D, D), lambda i: (0, 0)),
                  pl.BlockSpec((RB, 1), lambda i: (i, 0))],
        out_specs=pl.BlockSpec((RB, D), lambda i: (i, 0)),
        out_shape=jax.ShapeDtypeStruct((N, D), jnp.float32),
    )(x, W, dis)


def _layer_kernel(p0_ref, p1_ref, y_ref, dis_ref, b_ref, w_ref, o_ref):
    t = (p0_ref[...] + p1_ref[...] + y_ref[...]) * dis_ref[...] + b_ref[...]
    t = jnp.maximum(t, 0.0)
    h = jnp.dot(t, w_ref[...], preferred_element_type=jnp.float32)
    o_ref[...] = h * dis_ref[...]


def _layer(p0, p1, y, dis, b, W):
    return pl.pallas_call(
        _layer_kernel,
        grid=(N // RB,),
        in_specs=[pl.BlockSpec((RB, D), lambda i: (i, 0)),
                  pl.BlockSpec((RB, D), lambda i: (i, 0)),
                  pl.BlockSpec((RB, D), lambda i: (i, 0)),
                  pl.BlockSpec((RB, 1), lambda i: (i, 0)),
                  pl.BlockSpec((1, D), lambda i: (0, 0)),
                  pl.BlockSpec((D, D), lambda i: (0, 0))],
        out_specs=pl.BlockSpec((RB, D), lambda i: (i, 0)),
        out_shape=jax.ShapeDtypeStruct((N, D), jnp.float32),
    )(p0, p1, y, dis, b.reshape(1, D), W)


def _pool_kernel(p0_ref, p1_ref, y_ref, dis_ref, b_ref, batch_ref, o_ref,
                 acc, cnt):
    i = pl.program_id(0)

    @pl.when(i == 0)
    def _():
        acc[...] = jnp.zeros_like(acc)
        cnt[...] = jnp.zeros_like(cnt)

    h = (p0_ref[...] + p1_ref[...] + y_ref[...]) * dis_ref[...] + b_ref[...]
    sel = (batch_ref[...] == lax.broadcasted_iota(jnp.float32, (1, G), 1))
    sel = sel.astype(jnp.float32)
    acc[...] += lax.dot_general(sel, h, (((0,), (0,)), ((), ())),
                                preferred_element_type=jnp.float32)
    cnt[...] += lax.dot_general(sel, jnp.ones_like(h), (((0,), (0,)), ((), ())),
                                preferred_element_type=jnp.float32)

    @pl.when(i == pl.num_programs(0) - 1)
    def _():
        o_ref[...] = acc[...] / jnp.maximum(cnt[...], 1.0)


def _pool(p0, p1, y, dis, b, batchf):
    return pl.pallas_call(
        _pool_kernel,
        grid=(N // RB,),
        in_specs=[pl.BlockSpec((RB, D), lambda i: (i, 0)),
                  pl.BlockSpec((RB, D), lambda i: (i, 0)),
                  pl.BlockSpec((RB, D), lambda i: (i, 0)),
                  pl.BlockSpec((RB, 1), lambda i: (i, 0)),
                  pl.BlockSpec((1, D), lambda i: (0, 0)),
                  pl.BlockSpec((RB, 1), lambda i: (i, 0))],
        out_specs=pl.BlockSpec((G, D), lambda i: (0, 0)),
        out_shape=jax.ShapeDtypeStruct((G, D), jnp.float32),
        scratch_shapes=[pltpu.VMEM((G, D), jnp.float32),
                        pltpu.VMEM((G, D), jnp.float32)],
    )(p0, p1, y, dis, b.reshape(1, D), batchf)


def kernel(x, W1, b1, W2, b2, W3, b3, edge_index, batch):
    x = x.astype(jnp.float32)
    row2d = edge_index[0].reshape(NWIN, WE)
    col2d = edge_index[1].reshape(NWIN, WE)
    batchf = batch.astype(jnp.float32).reshape(N, 1)

    dsum = _sc_degree(col2d)
    dis = _dis(dsum[0, :N, 0:1], dsum[1, :N, 0:1])

    y1 = _mm_scale(x, W1, dis)
    p = _sc_scatter(y1, row2d, col2d)
    y2 = _layer(p[0, :N], p[1, :N], y1, dis, b1, W2)
    q = _sc_scatter(y2, row2d, col2d)
    y3 = _layer(q[0, :N], q[1, :N], y2, dis, b2, W3)
    r = _sc_scatter(y3, row2d, col2d)
    return _pool(r[0, :N], r[1, :N], y3, dis, b3, batchf)


# speedup vs baseline: 18.4542x; 18.4542x over previous
import dataclasses
import functools

import jax
import jax.numpy as jnp
from jax import lax
from jax.experimental import pallas as pl
from jax.experimental.pallas import tpu as pltpu
from jax.experimental.pallas import tpu_sc as plsc

N = 10000
E = 320000
G = 64
D = 128

NPAD = 10240
WE = 125
NWIN = E // WE
RPT = NWIN // 32
RB = 1000


def _vsc_mesh():
    return plsc.VectorSubcoreMesh(core_axis_name="c", subcore_axis_name="s")


def _sc_params():
    return dataclasses.replace(pltpu.CompilerParams(),
                               needs_layout_passes=False)


def _sc_degree(col16):
    NV = (E // 16) // 32
    STRIDE = NPAD // 16

    @functools.partial(
        pl.kernel,
        out_type=jax.ShapeDtypeStruct((2, NPAD), jnp.float32),
        mesh=_vsc_mesh(),
        scratch_types=[
            pltpu.VMEM((NV, 16), jnp.int32),
            pltpu.VMEM((NPAD,), jnp.float32),
            pltpu.VMEM((16, STRIDE), jnp.float32),
            pltpu.VMEM((STRIDE,), jnp.float32),
            pltpu.VMEM_SHARED((16, NPAD), jnp.float32),
        ],
        compiler_params=_sc_params(),
    )
    def k(col_hbm, out_hbm, idx_v, hist_v, rbuf, rout, hists_sh):
        c = lax.axis_index("c")
        s = lax.axis_index("s")
        wid = s * 2 + c

        @pl.loop(0, NPAD // 16)
        def _(i):
            hist_v[pl.ds(i * 16, 16)] = jnp.zeros((16,), jnp.float32)

        pltpu.sync_copy(col_hbm.at[wid], idx_v)

        @pl.loop(0, NV)
        def _(j):
            v = idx_v[j, :]
            vals, msk = plsc.scan_count(v)
            plsc.addupdate_scatter(hist_v, [v], vals.astype(jnp.float32),
                                   mask=msk)

        pltpu.sync_copy(hist_v, hists_sh.at[s])
        plsc.subcore_barrier()

        for t in range(16):
            pltpu.sync_copy(hists_sh.at[t, pl.ds(s * STRIDE, STRIDE)],
                            rbuf.at[t])

        @pl.loop(0, STRIDE // 16)
        def _(kk):
            a = rbuf[0, pl.ds(kk * 16, 16)]
            for t in range(1, 16):
                a = a + rbuf[t, pl.ds(kk * 16, 16)]
            rout[pl.ds(kk * 16, 16)] = a

        pltpu.sync_copy(rout, out_hbm.at[c, pl.ds(s * STRIDE, STRIDE)])

    return k(col16)


def _sc_scatter(y, row2d, col2d):
    @functools.partial(
        pl.kernel,
        out_type=jax.ShapeDtypeStruct((2, NPAD, D), jnp.float32),
        mesh=_vsc_mesh(),
        scratch_types=[
            pltpu.VMEM((RPT, WE), jnp.int32),
            pltpu.VMEM((RPT, WE), jnp.int32),
            pltpu.VMEM((WE, D), jnp.float32),
            pltpu.VMEM((64, D), jnp.float32),
            pltpu.VMEM_SHARED((NPAD, D), jnp.float32),
        ],
    )
    def k(y_hbm, row_hbm, col_hbm, out_hbm, row_v, col_v, gbuf, zbuf, accum):
        c = lax.axis_index("c")
        s = lax.axis_index("s")
        wid = s * 2 + c

        @pl.loop(0, 64)
        def _(i):
            @pl.loop(0, D // 16)
            def _(j):
                zbuf[i, pl.ds(j * 16, 16)] = jnp.zeros((16,), jnp.float32)

        @pl.loop(0, 10)
        def _(i):
            pltpu.sync_copy(zbuf, accum.at[pl.ds(s * 640 + i * 64, 64)])

        plsc.subcore_barrier()
        pltpu.sync_copy(row_hbm.at[pl.ds(wid * RPT, RPT)], row_v)
        pltpu.sync_copy(col_hbm.at[pl.ds(wid * RPT, RPT)], col_v)

        @pl.loop(0, RPT)
        def _(j):
            pltpu.sync_copy(y_hbm.at[row_v.at[j]], gbuf)
            pltpu.sync_copy(gbuf, accum.at[col_v.at[j]], add=True)

        plsc.subcore_barrier()
        pltpu.sync_copy(accum.at[pl.ds(s * 640, 640)],
                        out_hbm.at[c, pl.ds(s * 640, 640)])

    return k(y, row2d, col2d)


def _dis_kernel(d0_ref, d1_ref, o_ref):
    o_ref[...] = lax.rsqrt(1.0 + d0_ref[...] + d1_ref[...])


def _dis(d0, d1):
    return pl.pallas_call(
        _dis_kernel,
        grid=(N // RB,),
        in_specs=[pl.BlockSpec((RB, 1), lambda i: (i, 0)),
                  pl.BlockSpec((RB, 1), lambda i: (i, 0))],
        out_specs=pl.BlockSpec((RB, 1), lambda i: (i, 0)),
        out_shape=jax.ShapeDtypeStruct((N, 1), jnp.float32),
    )(d0, d1)


def _mm_scale_kernel(x_ref, w_ref, dis_ref, o_ref):
    h = jnp.dot(x_ref[...], w_ref[...], preferred_element_type=jnp.float32)
    o_ref[...] = h * dis_ref[...]


def _mm_scale(x, W, dis):
    return pl.pallas_call(
        _mm_scale_kernel,
        grid=(N // RB,),
        in_specs=[pl.BlockSpec((RB, D), lambda i: (i, 0)),
                  pl.BlockSpec((D, D), lambda i: (0, 0)),
                  pl.BlockSpec((RB, 1), lambda i: (i, 0))],
        out_specs=pl.BlockSpec((RB, D), lambda i: (i, 0)),
        out_shape=jax.ShapeDtypeStruct((N, D), jnp.float32),
    )(x, W, dis)


def _layer_kernel(p0_ref, p1_ref, y_ref, dis_ref, b_ref, w_ref, o_ref):
    t = (p0_ref[...] + p1_ref[...] + y_ref[...]) * dis_ref[...] + b_ref[...]
    t = jnp.maximum(t, 0.0)
    h = jnp.dot(t, w_ref[...], preferred_element_type=jnp.float32)
    o_ref[...] = h * dis_ref[...]


def _layer(p0, p1, y, dis, b, W):
    return pl.pallas_call(
        _layer_kernel,
        grid=(N // RB,),
        in_specs=[pl.BlockSpec((RB, D), lambda i: (i, 0)),
                  pl.BlockSpec((RB, D), lambda i: (i, 0)),
                  pl.BlockSpec((RB, D), lambda i: (i, 0)),
                  pl.BlockSpec((RB, 1), lambda i: (i, 0)),
                  pl.BlockSpec((1, D), lambda i: (0, 0)),
                  pl.BlockSpec((D, D), lambda i: (0, 0))],
        out_specs=pl.BlockSpec((RB, D), lambda i: (i, 0)),
        out_shape=jax.ShapeDtypeStruct((N, D), jnp.float32),
    )(p0, p1, y, dis, b.reshape(1, D), W)


def _pool_kernel(p0_ref, p1_ref, y_ref, dis_ref, b_ref, batch_ref, o_ref,
                 acc, cnt):
    i = pl.program_id(0)

    @pl.when(i == 0)
    def _():
        acc[...] = jnp.zeros_like(acc)
        cnt[...] = jnp.zeros_like(cnt)

    h = (p0_ref[...] + p1_ref[...] + y_ref[...]) * dis_ref[...] + b_ref[...]
    gid = lax.broadcasted_iota(jnp.int32, (1, G), 1).astype(jnp.float32)
    sel = (batch_ref[...] == gid).astype(jnp.float32)
    acc[...] += lax.dot_general(sel, h, (((0,), (0,)), ((), ())),
                                preferred_element_type=jnp.float32)
    cnt[...] += lax.dot_general(sel, jnp.ones_like(h), (((0,), (0,)), ((), ())),
                                preferred_element_type=jnp.float32)

    @pl.when(i == pl.num_programs(0) - 1)
    def _():
        o_ref[...] = acc[...] / jnp.maximum(cnt[...], 1.0)


def _pool(p0, p1, y, dis, b, batchf):
    return pl.pallas_call(
        _pool_kernel,
        grid=(N // RB,),
        in_specs=[pl.BlockSpec((RB, D), lambda i: (i, 0)),
                  pl.BlockSpec((RB, D), lambda i: (i, 0)),
                  pl.BlockSpec((RB, D), lambda i: (i, 0)),
                  pl.BlockSpec((RB, 1), lambda i: (i, 0)),
                  pl.BlockSpec((1, D), lambda i: (0, 0)),
                  pl.BlockSpec((RB, 1), lambda i: (i, 0))],
        out_specs=pl.BlockSpec((G, D), lambda i: (0, 0)),
        out_shape=jax.ShapeDtypeStruct((G, D), jnp.float32),
        scratch_shapes=[pltpu.VMEM((G, D), jnp.float32),
                        pltpu.VMEM((G, D), jnp.float32)],
    )(p0, p1, y, dis, b.reshape(1, D), batchf)


def kernel(x, W1, b1, W2, b2, W3, b3, edge_index, batch):
    x = x.astype(jnp.float32)
    row2d = edge_index[0].reshape(NWIN, WE)
    col2d = edge_index[1].reshape(NWIN, WE)
    col16 = edge_index[1].reshape(32, E // (16 * 32), 16)
    batchf = batch.astype(jnp.float32).reshape(N, 1)

    dsum = _sc_degree(col16)
    dis = _dis(dsum[0, :N].reshape(N, 1), dsum[1, :N].reshape(N, 1))

    y1 = _mm_scale(x, W1, dis)
    p = _sc_scatter(y1, row2d, col2d)
    y2 = _layer(p[0, :N], p[1, :N], y1, dis, b1, W2)
    q = _sc_scatter(y2, row2d, col2d)
    y3 = _layer(q[0, :N], q[1, :N], y2, dis, b2, W3)
    r = _sc_scatter(y3, row2d, col2d)
    return _pool(r[0, :N], r[1, :N], y3, dis, b3, batchf)


# speedup vs baseline: 20.7813x; 1.1261x over previous
import dataclasses
import functools

import jax
import jax.numpy as jnp
from jax import lax
from jax.experimental import pallas as pl
from jax.experimental.pallas import tpu as pltpu
from jax.experimental.pallas import tpu_sc as plsc

N = 10000
E = 320000
G = 64
D = 128

NPAD = 10240
WE = 96
RPT = 108
HALF = RPT // 2
EPAD = 32 * RPT * WE
RB = 1000


def _vsc_mesh():
    return plsc.VectorSubcoreMesh(core_axis_name="c", subcore_axis_name="s")


def _sc_params():
    return dataclasses.replace(pltpu.CompilerParams(),
                               needs_layout_passes=False)


def _sc_degree(col16):
    NV = (E // 16) // 32
    STRIDE = NPAD // 16

    @functools.partial(
        pl.kernel,
        out_type=jax.ShapeDtypeStruct((2, NPAD), jnp.float32),
        mesh=_vsc_mesh(),
        scratch_types=[
            pltpu.VMEM((NV, 16), jnp.int32),
            pltpu.VMEM((NPAD,), jnp.float32),
            pltpu.VMEM((16, STRIDE), jnp.float32),
            pltpu.VMEM((STRIDE,), jnp.float32),
            pltpu.VMEM_SHARED((16, NPAD), jnp.float32),
        ],
        compiler_params=_sc_params(),
    )
    def k(col_hbm, out_hbm, idx_v, hist_v, rbuf, rout, hists_sh):
        c = lax.axis_index("c")
        s = lax.axis_index("s")
        wid = s * 2 + c

        @pl.loop(0, NPAD // 16)
        def _(i):
            hist_v[pl.ds(i * 16, 16)] = jnp.zeros((16,), jnp.float32)

        pltpu.sync_copy(col_hbm.at[wid], idx_v)

        @pl.loop(0, NV)
        def _(j):
            v = idx_v[j, :]
            vals, msk = plsc.scan_count(v)
            plsc.addupdate_scatter(hist_v, [v], vals.astype(jnp.float32),
                                   mask=msk)

        pltpu.sync_copy(hist_v, hists_sh.at[s])
        plsc.subcore_barrier()

        for t in range(16):
            pltpu.sync_copy(hists_sh.at[t, pl.ds(s * STRIDE, STRIDE)],
                            rbuf.at[t])

        @pl.loop(0, STRIDE // 16)
        def _(kk):
            a = rbuf[0, pl.ds(kk * 16, 16)]
            for t in range(1, 16):
                a = a + rbuf[t, pl.ds(kk * 16, 16)]
            rout[pl.ds(kk * 16, 16)] = a

        pltpu.sync_copy(rout, out_hbm.at[c, pl.ds(s * STRIDE, STRIDE)])

    return k(col16)


def _sc_scatter(y, row2d, col2d):
    @functools.partial(
        pl.kernel,
        out_type=jax.ShapeDtypeStruct((2, NPAD, D), jnp.float32),
        mesh=_vsc_mesh(),
        scratch_types=[
            pltpu.VMEM((HALF, WE), jnp.int32),
            pltpu.VMEM((HALF, WE), jnp.int32),
            pltpu.VMEM((WE, D), jnp.float32),
            pltpu.VMEM((WE, D), jnp.float32),
            pltpu.VMEM_SHARED((NPAD, D), jnp.float32),
            pltpu.SemaphoreType.DMA,
            pltpu.SemaphoreType.DMA,
        ],
    )
    def k(y_hbm, row_hbm, col_hbm, out_hbm, row_v, col_v, gbuf0, gbuf1,
          accum, gsem0, gsem1):
        c = lax.axis_index("c")
        s = lax.axis_index("s")
        wid = s * 2 + c

        @pl.loop(0, 16)
        def _(i):
            @pl.loop(0, D // 16)
            def _(j):
                gbuf0[i, pl.ds(j * 16, 16)] = jnp.zeros((16,), jnp.float32)

        @pl.loop(0, 40)
        def _(i):
            pltpu.sync_copy(gbuf0.at[pl.ds(0, 16)],
                            accum.at[pl.ds(s * 640 + i * 16, 16)])

        plsc.subcore_barrier()

        for h in range(2):
            pltpu.sync_copy(row_hbm.at[wid, h], row_v)
            pltpu.sync_copy(col_hbm.at[wid, h], col_v)
            pltpu.async_copy(y_hbm.at[row_v.at[0]], gbuf0, gsem0)

            @pl.loop(0, HALF // 2)
            def _(i):
                j0 = 2 * i
                pltpu.make_async_copy(y_hbm.at[row_v.at[j0]], gbuf0,
                                      gsem0).wait()
                pltpu.async_copy(y_hbm.at[row_v.at[j0 + 1]], gbuf1, gsem1)
                pltpu.sync_copy(gbuf0, accum.at[col_v.at[j0]], add=True)
                pltpu.make_async_copy(y_hbm.at[row_v.at[j0 + 1]], gbuf1,
                                      gsem1).wait()

                @pl.when(j0 + 2 < HALF)
                def _():
                    pltpu.async_copy(y_hbm.at[row_v.at[j0 + 2]], gbuf0, gsem0)

                pltpu.sync_copy(gbuf1, accum.at[col_v.at[j0 + 1]], add=True)

        plsc.subcore_barrier()
        pltpu.sync_copy(accum.at[pl.ds(s * 640, 640)],
                        out_hbm.at[c, pl.ds(s * 640, 640)])

    return k(y, row2d, col2d)


def _dis_kernel(d0_ref, d1_ref, o_ref):
    o_ref[...] = lax.rsqrt(1.0 + d0_ref[...] + d1_ref[...])


def _dis(d0, d1):
    return pl.pallas_call(
        _dis_kernel,
        grid=(N // RB,),
        in_specs=[pl.BlockSpec((RB, 1), lambda i: (i, 0)),
                  pl.BlockSpec((RB, 1), lambda i: (i, 0))],
        out_specs=pl.BlockSpec((RB, 1), lambda i: (i, 0)),
        out_shape=jax.ShapeDtypeStruct((N, 1), jnp.float32),
    )(d0, d1)


def _mm_scale_kernel(x_ref, w_ref, dis_ref, o_ref):
    h = jnp.dot(x_ref[...], w_ref[...], preferred_element_type=jnp.float32)
    o_ref[...] = h * dis_ref[...]


def _mm_scale(x, W, dis):
    return pl.pallas_call(
        _mm_scale_kernel,
        grid=(N // RB,),
        in_specs=[pl.BlockSpec((RB, D), lambda i: (i, 0)),
                  pl.BlockSpec((D, D), lambda i: (0, 0)),
                  pl.BlockSpec((RB, 1), lambda i: (i, 0))],
        out_specs=pl.BlockSpec((RB, D), lambda i: (i, 0)),
        out_shape=jax.ShapeDtypeStruct((N, D), jnp.float32),
    )(x, W, dis)


def _layer_kernel(p0_ref, p1_ref, y_ref, dis_ref, b_ref, w_ref, o_ref):
    t = (p0_ref[...] + p1_ref[...] + y_ref[...]) * dis_ref[...] + b_ref[...]
    t = jnp.maximum(t, 0.0)
    h = jnp.dot(t, w_ref[...], preferred_element_type=jnp.float32)
    o_ref[...] = h * dis_ref[...]


def _layer(p0, p1, y, dis, b, W):
    return pl.pallas_call(
        _layer_kernel,
        grid=(N // RB,),
        in_specs=[pl.BlockSpec((RB, D), lambda i: (i, 0)),
                  pl.BlockSpec((RB, D), lambda i: (i, 0)),
                  pl.BlockSpec((RB, D), lambda i: (i, 0)),
                  pl.BlockSpec((RB, 1), lambda i: (i, 0)),
                  pl.BlockSpec((1, D), lambda i: (0, 0)),
                  pl.BlockSpec((D, D), lambda i: (0, 0))],
        out_specs=pl.BlockSpec((RB, D), lambda i: (i, 0)),
        out_shape=jax.ShapeDtypeStruct((N, D), jnp.float32),
    )(p0, p1, y, dis, b.reshape(1, D), W)


def _pool_kernel(p0_ref, p1_ref, y_ref, dis_ref, b_ref, batch_ref, o_ref,
                 acc, cnt):
    i = pl.program_id(0)

    @pl.when(i == 0)
    def _():
        acc[...] = jnp.zeros_like(acc)
        cnt[...] = jnp.zeros_like(cnt)

    h = (p0_ref[...] + p1_ref[...] + y_ref[...]) * dis_ref[...] + b_ref[...]
    gid = lax.broadcasted_iota(jnp.int32, (1, G), 1).astype(jnp.float32)
    sel = (batch_ref[...] == gid).astype(jnp.float32)
    acc[...] += lax.dot_general(sel, h, (((0,), (0,)), ((), ())),
                                preferred_element_type=jnp.float32)
    cnt[...] += lax.dot_general(sel, jnp.ones_like(h), (((0,), (0,)), ((), ())),
                                preferred_element_type=jnp.float32)

    @pl.when(i == pl.num_programs(0) - 1)
    def _():
        o_ref[...] = acc[...] / jnp.maximum(cnt[...], 1.0)


def _pool(p0, p1, y, dis, b, batchf):
    return pl.pallas_call(
        _pool_kernel,
        grid=(N // RB,),
        in_specs=[pl.BlockSpec((RB, D), lambda i: (i, 0)),
                  pl.BlockSpec((RB, D), lambda i: (i, 0)),
                  pl.BlockSpec((RB, D), lambda i: (i, 0)),
                  pl.BlockSpec((RB, 1), lambda i: (i, 0)),
                  pl.BlockSpec((1, D), lambda i: (0, 0)),
                  pl.BlockSpec((RB, 1), lambda i: (i, 0))],
        out_specs=pl.BlockSpec((G, D), lambda i: (0, 0)),
        out_shape=jax.ShapeDtypeStruct((G, D), jnp.float32),
        scratch_shapes=[pltpu.VMEM((G, D), jnp.float32),
                        pltpu.VMEM((G, D), jnp.float32)],
    )(p0, p1, y, dis, b.reshape(1, D), batchf)


def kernel(x, W1, b1, W2, b2, W3, b3, edge_index, batch):
    x = x.astype(jnp.float32)
    npad_e = EPAD - E
    pad_row = (jnp.arange(npad_e, dtype=jnp.int32) % 1024)
    pad_col = N + (jnp.arange(npad_e, dtype=jnp.int32) % (NPAD - N))
    row2d = jnp.concatenate([edge_index[0], pad_row]).reshape(32, 2, HALF, WE)
    col2d = jnp.concatenate([edge_index[1], pad_col]).reshape(32, 2, HALF, WE)
    col16 = edge_index[1].reshape(32, E // (16 * 32), 16)
    batchf = batch.astype(jnp.float32).reshape(N, 1)

    dsum = _sc_degree(col16)
    dis = _dis(dsum[0, :N].reshape(N, 1), dsum[1, :N].reshape(N, 1))

    y1 = _mm_scale(x, W1, dis)
    p = _sc_scatter(y1, row2d, col2d)
    y2 = _layer(p[0, :N], p[1, :N], y1, dis, b1, W2)
    q = _sc_scatter(y2, row2d, col2d)
    y3 = _layer(q[0, :N], q[1, :N], y2, dis, b2, W3)
    r = _sc_scatter(y3, row2d, col2d)
    return _pool(r[0, :N], r[1, :N], y3, dis, b3, batchf)


# speedup vs baseline: 23.3060x; 1.1215x over previous
import dataclasses
import functools

import jax
import jax.numpy as jnp
from jax import lax
from jax.experimental import pallas as pl
from jax.experimental.pallas import tpu as pltpu
from jax.experimental.pallas import tpu_sc as plsc

N = 10000
E = 320000
G = 64
D = 128

NPAD = 10240
WE = 128
RPT = 80
NCHUNK = 4
CHW = RPT // NCHUNK
EPAD = 32 * RPT * WE
RB = 1000


def _vsc_mesh():
    return plsc.VectorSubcoreMesh(core_axis_name="c", subcore_axis_name="s")


def _sc_params():
    return dataclasses.replace(pltpu.CompilerParams(),
                               needs_layout_passes=False)


def _sc_degree(col16):
    NV = (E // 16) // 32
    STRIDE = NPAD // 16

    @functools.partial(
        pl.kernel,
        out_type=jax.ShapeDtypeStruct((2, NPAD), jnp.float32),
        mesh=_vsc_mesh(),
        scratch_types=[
            pltpu.VMEM((NV, 16), jnp.int32),
            pltpu.VMEM((NPAD,), jnp.float32),
            pltpu.VMEM((16, STRIDE), jnp.float32),
            pltpu.VMEM((STRIDE,), jnp.float32),
            pltpu.VMEM_SHARED((16, NPAD), jnp.float32),
        ],
        compiler_params=_sc_params(),
    )
    def k(col_hbm, out_hbm, idx_v, hist_v, rbuf, rout, hists_sh):
        c = lax.axis_index("c")
        s = lax.axis_index("s")
        wid = s * 2 + c

        @pl.loop(0, NPAD // 16)
        def _(i):
            hist_v[pl.ds(i * 16, 16)] = jnp.zeros((16,), jnp.float32)

        pltpu.sync_copy(col_hbm.at[wid], idx_v)

        @pl.loop(0, NV)
        def _(j):
            v = idx_v[j, :]
            vals, msk = plsc.scan_count(v)
            plsc.addupdate_scatter(hist_v, [v], vals.astype(jnp.float32),
                                   mask=msk)

        pltpu.sync_copy(hist_v, hists_sh.at[s])
        plsc.subcore_barrier()

        for t in range(16):
            pltpu.sync_copy(hists_sh.at[t, pl.ds(s * STRIDE, STRIDE)],
                            rbuf.at[t])

        @pl.loop(0, STRIDE // 16)
        def _(kk):
            a = rbuf[0, pl.ds(kk * 16, 16)]
            for t in range(1, 16):
                a = a + rbuf[t, pl.ds(kk * 16, 16)]
            rout[pl.ds(kk * 16, 16)] = a

        pltpu.sync_copy(rout, out_hbm.at[c, pl.ds(s * STRIDE, STRIDE)])

    return k(col16)


def _sc_scatter(y, row2d, col2d):
    @functools.partial(
        pl.kernel,
        out_type=jax.ShapeDtypeStruct((2, NPAD, D), jnp.float32),
        mesh=_vsc_mesh(),
        scratch_types=[
            pltpu.VMEM((CHW, WE), jnp.int32),
            pltpu.VMEM((CHW, WE), jnp.int32),
            pltpu.VMEM((WE, D), jnp.float32),
            pltpu.VMEM((WE, D), jnp.float32),
            pltpu.VMEM_SHARED((NPAD, D), jnp.float32),
            pltpu.SemaphoreType.DMA,
            pltpu.SemaphoreType.DMA,
        ],
    )
    def k(y_hbm, row_hbm, col_hbm, out_hbm, row_v, col_v, gbuf0, gbuf1,
          accum, gsem0, gsem1):
        c = lax.axis_index("c")
        s = lax.axis_index("s")
        wid = s * 2 + c

        @pl.loop(0, 16)
        def _(i):
            @pl.loop(0, D // 16)
            def _(j):
                gbuf0[i, pl.ds(j * 16, 16)] = jnp.zeros((16,), jnp.float32)

        @pl.loop(0, 40)
        def _(i):
            pltpu.sync_copy(gbuf0.at[pl.ds(0, 16)],
                            accum.at[pl.ds(s * 640 + i * 16, 16)])

        plsc.subcore_barrier()

        @pl.loop(0, NCHUNK)
        def _(h):
            pltpu.sync_copy(row_hbm.at[wid, h], row_v)
            pltpu.sync_copy(col_hbm.at[wid, h], col_v)
            pltpu.async_copy(y_hbm.at[row_v.at[0]], gbuf0, gsem0)

            @pl.loop(0, CHW // 2)
            def _(i):
                j0 = 2 * i
                pltpu.make_async_copy(y_hbm.at[row_v.at[j0]], gbuf0,
                                      gsem0).wait()
                pltpu.async_copy(y_hbm.at[row_v.at[j0 + 1]], gbuf1, gsem1)
                pltpu.sync_copy(gbuf0, accum.at[col_v.at[j0]], add=True)
                pltpu.make_async_copy(y_hbm.at[row_v.at[j0 + 1]], gbuf1,
                                      gsem1).wait()

                @pl.when(j0 + 2 < CHW)
                def _():
                    pltpu.async_copy(y_hbm.at[row_v.at[j0 + 2]], gbuf0, gsem0)

                pltpu.sync_copy(gbuf1, accum.at[col_v.at[j0 + 1]], add=True)

        plsc.subcore_barrier()
        pltpu.sync_copy(accum.at[pl.ds(s * 640, 640)],
                        out_hbm.at[c, pl.ds(s * 640, 640)])

    return k(y, row2d, col2d)


def _mm_scale_kernel(x_ref, w_ref, d0_ref, d1_ref, o_ref):
    dis = lax.rsqrt(1.0 + d0_ref[...] + d1_ref[...])
    h = jnp.dot(x_ref[...], w_ref[...], preferred_element_type=jnp.float32)
    o_ref[...] = h * dis


def _mm_scale(x, W, d0, d1):
    return pl.pallas_call(
        _mm_scale_kernel,
        grid=(N // RB,),
        in_specs=[pl.BlockSpec((RB, D), lambda i: (i, 0)),
                  pl.BlockSpec((D, D), lambda i: (0, 0)),
                  pl.BlockSpec((RB, 1), lambda i: (i, 0)),
                  pl.BlockSpec((RB, 1), lambda i: (i, 0))],
        out_specs=pl.BlockSpec((RB, D), lambda i: (i, 0)),
        out_shape=jax.ShapeDtypeStruct((N, D), jnp.float32),
    )(x, W, d0, d1)


def _layer_kernel(p0_ref, p1_ref, y_ref, d0_ref, d1_ref, b_ref, w_ref, o_ref):
    dis = lax.rsqrt(1.0 + d0_ref[...] + d1_ref[...])
    t = (p0_ref[...] + p1_ref[...] + y_ref[...]) * dis + b_ref[...]
    t = jnp.maximum(t, 0.0)
    h = jnp.dot(t, w_ref[...], preferred_element_type=jnp.float32)
    o_ref[...] = h * dis


def _layer(p0, p1, y, d0, d1, b, W):
    return pl.pallas_call(
        _layer_kernel,
        grid=(N // RB,),
        in_specs=[pl.BlockSpec((RB, D), lambda i: (i, 0)),
                  pl.BlockSpec((RB, D), lambda i: (i, 0)),
                  pl.BlockSpec((RB, D), lambda i: (i, 0)),
                  pl.BlockSpec((RB, 1), lambda i: (i, 0)),
                  pl.BlockSpec((RB, 1), lambda i: (i, 0)),
                  pl.BlockSpec((1, D), lambda i: (0, 0)),
                  pl.BlockSpec((D, D), lambda i: (0, 0))],
        out_specs=pl.BlockSpec((RB, D), lambda i: (i, 0)),
        out_shape=jax.ShapeDtypeStruct((N, D), jnp.float32),
    )(p0, p1, y, d0, d1, b.reshape(1, D), W)


def _pool_kernel(p0_ref, p1_ref, y_ref, d0_ref, d1_ref, b_ref, batch_ref,
                 o_ref, acc, cnt):
    i = pl.program_id(0)

    @pl.when(i == 0)
    def _():
        acc[...] = jnp.zeros_like(acc)
        cnt[...] = jnp.zeros_like(cnt)

    dis = lax.rsqrt(1.0 + d0_ref[...] + d1_ref[...])
    h = (p0_ref[...] + p1_ref[...] + y_ref[...]) * dis + b_ref[...]
    gid = lax.broadcasted_iota(jnp.int32, (1, G), 1).astype(jnp.float32)
    sel = (batch_ref[...] == gid).astype(jnp.float32)
    acc[...] += lax.dot_general(sel, h, (((0,), (0,)), ((), ())),
                                preferred_element_type=jnp.float32)
    cnt[...] += lax.dot_general(sel, jnp.ones_like(h), (((0,), (0,)), ((), ())),
                                preferred_element_type=jnp.float32)

    @pl.when(i == pl.num_programs(0) - 1)
    def _():
        o_ref[...] = acc[...] / jnp.maximum(cnt[...], 1.0)


def _pool(p0, p1, y, d0, d1, b, batchf):
    return pl.pallas_call(
        _pool_kernel,
        grid=(N // RB,),
        in_specs=[pl.BlockSpec((RB, D), lambda i: (i, 0)),
                  pl.BlockSpec((RB, D), lambda i: (i, 0)),
                  pl.BlockSpec((RB, D), lambda i: (i, 0)),
                  pl.BlockSpec((RB, 1), lambda i: (i, 0)),
                  pl.BlockSpec((RB, 1), lambda i: (i, 0)),
                  pl.BlockSpec((1, D), lambda i: (0, 0)),
                  pl.BlockSpec((RB, 1), lambda i: (i, 0))],
        out_specs=pl.BlockSpec((G, D), lambda i: (0, 0)),
        out_shape=jax.ShapeDtypeStruct((G, D), jnp.float32),
        scratch_shapes=[pltpu.VMEM((G, D), jnp.float32),
                        pltpu.VMEM((G, D), jnp.float32)],
    )(p0, p1, y, d0, d1, b.reshape(1, D), batchf)


def kernel(x, W1, b1, W2, b2, W3, b3, edge_index, batch):
    x = x.astype(jnp.float32)
    npad_e = EPAD - E
    pad_row = (jnp.arange(npad_e, dtype=jnp.int32) % 1024)
    pad_col = N + (jnp.arange(npad_e, dtype=jnp.int32) % (NPAD - N))
    row2d = jnp.concatenate([edge_index[0], pad_row]).reshape(32, NCHUNK,
                                                              CHW, WE)
    col2d = jnp.concatenate([edge_index[1], pad_col]).reshape(32, NCHUNK,
                                                              CHW, WE)
    col16 = edge_index[1].reshape(32, E // (16 * 32), 16)
    batchf = batch.astype(jnp.float32).reshape(N, 1)

    dsum = _sc_degree(col16)
    d0 = dsum[0, :N].reshape(N, 1)
    d1 = dsum[1, :N].reshape(N, 1)

    y1 = _mm_scale(x, W1, d0, d1)
    p = _sc_scatter(y1, row2d, col2d)
    y2 = _layer(p[0, :N], p[1, :N], y1, d0, d1, b1, W2)
    q = _sc_scatter(y2, row2d, col2d)
    y3 = _layer(q[0, :N], q[1, :N], y2, d0, d1, b2, W3)
    r = _sc_scatter(y3, row2d, col2d)
    return _pool(r[0, :N], r[1, :N], y3, d0, d1, b3, batchf)
